# baseline (speedup 1.0000x reference)
import jax
import jax.numpy as jnp
from jax import lax
from jax.experimental import pallas as pl
from jax.experimental.pallas import tpu as pltpu
from jax.experimental.pallas import tpu_sc as plsc

NUM_FACTORS = 64
NUM_ROWS = 100000
BATCH = 16384
L = 16
NC = 2
NS = 16
NW = NC * NS
MAIN = 99968
TAILPAD = 128
ROWBUF = MAIN + TAILPAD
CH = 4096
NCH = BATCH // CH
PR = 128
CROWS = CH // 128

_COMPILER_PARAMS = pltpu.CompilerParams(
    needs_layout_passes=False, use_tc_tiling_on_sc=True)


def _pass1_kernel(uft_hbm, ift_hbm, utail_hbm, itail_hbm, uid_hbm, iid_hbm,
                  w_hbm, accs_hbm,
                  rowbuf, prod_v, ids0_v, ids1_v, w_v, zbuf_v, idxrows_v,
                  acc_sh, sem_m, sem_t, sem_i0, sem_i1):
    sid = lax.axis_index("s")
    cid = lax.axis_index("c")
    wid = sid * NC + cid
    pltpu.sync_copy(w_hbm, w_v)

    zero = jnp.zeros((L,), jnp.float32)
    iota = lax.iota(jnp.int32, L)
    for r in range(8):
        for c in range(8):
            zbuf_v[r, pl.ds(c * L, L)] = zero
    for h in range(NCH):
        for j in range(2):
            idxrows_v[h, pl.ds(j * L, L)] = iota + (h * CROWS + j * L)
    pltpu.sync_copy(zbuf_v, acc_sh.at[pl.ds(sid * 8, 8), :])
    plsc.subcore_barrier()

    idbufs = [(ids0_v, sem_i0), (ids1_v, sem_i1)]

    def load_row(f, table, tail, ids_hbm):
        cm = pltpu.async_copy(table.at[f, pl.ds(0, MAIN)],
                              rowbuf.at[pl.ds(0, MAIN)], sem_m)
        ct = pltpu.async_copy(tail.at[f, :],
                              rowbuf.at[pl.ds(MAIN, TAILPAD)], sem_t)
        pltpu.async_copy(ids_hbm.at[pl.ds(0, CH)], idbufs[0][0],
                         idbufs[0][1])
        cm.wait()
        ct.wait()

    def chunks(ids_hbm, body):
        for h in range(NCH):
            ids_v, sem_i = idbufs[h % 2]
            if h + 1 < NCH:
                nxt_v, nxt_s = idbufs[(h + 1) % 2]
                pltpu.async_copy(ids_hbm.at[pl.ds((h + 1) * CH, CH)],
                                 nxt_v, nxt_s)
            pltpu.make_async_copy(ids_hbm.at[pl.ds(0, CH)], ids_v,
                                  sem_i).wait()
            body(h, ids_v)

    def factor(f):
        load_row(f, uft_hbm, utail_hbm, uid_hbm)
        wspl = plsc.load_gather(w_v, [jnp.broadcast_to(f, (L,))])

        def ubody(h, ids_v):
            @plsc.parallel_loop(0, CH // L, unroll=8)
            def _g(v):
                idxv = ids_v[pl.ds(v * L, L)]
                r = h * CROWS + lax.shift_right_logical(v, 3)
                c = (v & 7) * L
                prod_v[r, pl.ds(c, L)] = (
                    plsc.load_gather(rowbuf, [idxv]) * wspl)
        chunks(uid_hbm, ubody)

        load_row(f, ift_hbm, itail_hbm, iid_hbm)

        def ibody(h, ids_v):
            @plsc.parallel_loop(0, CH // L, unroll=8)
            def _g(v):
                idxv = ids_v[pl.ds(v * L, L)]
                r = h * CROWS + lax.shift_right_logical(v, 3)
                c = (v & 7) * L
                prod_v[r, pl.ds(c, L)] = (
                    prod_v[r, pl.ds(c, L)]
                    * plsc.load_gather(rowbuf, [idxv]))
            pltpu.sync_copy(prod_v.at[pl.ds(h * CROWS, CROWS), :],
                            acc_sh.at[idxrows_v.at[h]], add=True)
        chunks(iid_hbm, ibody)

    factor(wid)
    factor(wid + 32)

    plsc.subcore_barrier()

    @pl.when(sid == 0)
    def _writeout():
        pltpu.sync_copy(acc_sh, accs_hbm.at[cid])


@jax.jit
def _run(user_ids, item_ids, user_factors, item_factors, w_vec, b):
    mesh = plsc.VectorSubcoreMesh(core_axis_name="c", subcore_axis_name="s")
    uft = user_factors.T
    ift = item_factors.T
    utail = jnp.pad(lax.slice(uft, (0, MAIN), (NUM_FACTORS, NUM_ROWS)),
                    ((0, 0), (0, TAILPAD - (NUM_ROWS - MAIN))))
    itail = jnp.pad(lax.slice(ift, (0, MAIN), (NUM_FACTORS, NUM_ROWS)),
                    ((0, 0), (0, TAILPAD - (NUM_ROWS - MAIN))))

    p1 = pl.kernel(
        _pass1_kernel,
        mesh=mesh,
        compiler_params=_COMPILER_PARAMS,
        out_type=jax.ShapeDtypeStruct((NC, PR, 128), jnp.float32),
        scratch_types=[
            pltpu.VMEM((ROWBUF,), jnp.float32),
            pltpu.VMEM((PR, 128), jnp.float32),
            pltpu.VMEM((CH,), jnp.int32),
            pltpu.VMEM((CH,), jnp.int32),
            pltpu.VMEM((NUM_FACTORS,), jnp.float32),
            pltpu.VMEM((8, 128), jnp.float32),
            pltpu.VMEM((NCH, CROWS), jnp.int32),
            pltpu.VMEM_SHARED((PR, 128), jnp.float32),
            pltpu.SemaphoreType.DMA,
            pltpu.SemaphoreType.DMA,
            pltpu.SemaphoreType.DMA,
            pltpu.SemaphoreType.DMA,
        ],
    )
    accs = p1(uft, ift, utail, itail, user_ids, item_ids, w_vec)
    return (accs[0] + accs[1]).reshape(BATCH) + b[0]


def kernel(user_ids, item_ids, user_factors, item_factors, W, b):
    uid = user_ids.astype(jnp.int32)
    iid = item_ids.astype(jnp.int32)
    w_vec = W.reshape(NUM_FACTORS).astype(jnp.float32)
    out = _run(uid, iid, user_factors, item_factors, w_vec,
               b.astype(jnp.float32))
    return out.reshape(BATCH, 1)


# speedup vs baseline: 1.0049x; 1.0049x over previous
import jax
import jax.numpy as jnp
from jax import lax
from jax.experimental import pallas as pl
from jax.experimental.pallas import tpu as pltpu
from jax.experimental.pallas import tpu_sc as plsc

NUM_FACTORS = 64
NUM_ROWS = 100000
BATCH = 16384
L = 16
NC = 2
NS = 16
NW = NC * NS
MAIN = 99968
TAILPAD = 128
ROWBUF = MAIN + TAILPAD
CH = 4096
NCH = BATCH // CH
PR = 128
CROWS = CH // 128

_COMPILER_PARAMS = pltpu.CompilerParams(
    needs_layout_passes=False, use_tc_tiling_on_sc=True)


def _pass1_kernel(uft_hbm, ift_hbm, utail_hbm, itail_hbm, uid_hbm, iid_hbm,
                  w_hbm, accs_hbm,
                  rowbuf, prod_v, ids0_v, ids1_v, w_v, zbuf_v, idxrows_v,
                  acc_sh, sem_m, sem_t, sem_i0, sem_i1):
    sid = lax.axis_index("s")
    cid = lax.axis_index("c")
    wid = sid * NC + cid
    pltpu.sync_copy(w_hbm, w_v)

    zero = jnp.zeros((L,), jnp.float32)
    iota = lax.iota(jnp.int32, L)
    for r in range(8):
        for c in range(8):
            zbuf_v[r, pl.ds(c * L, L)] = zero
    for h in range(NCH):
        for j in range(2):
            idxrows_v[h, pl.ds(j * L, L)] = iota + (h * CROWS + j * L)
    pltpu.sync_copy(zbuf_v, acc_sh.at[pl.ds(sid * 8, 8), :])
    plsc.subcore_barrier()

    idbufs = [(ids0_v, sem_i0), (ids1_v, sem_i1)]

    def load_row(f, table, tail, ids_hbm):
        cm = pltpu.async_copy(table.at[f, pl.ds(0, MAIN)],
                              rowbuf.at[pl.ds(0, MAIN)], sem_m)
        ct = pltpu.async_copy(tail.at[f, :],
                              rowbuf.at[pl.ds(MAIN, TAILPAD)], sem_t)
        pltpu.async_copy(ids_hbm.at[pl.ds(0, CH)], idbufs[0][0],
                         idbufs[0][1])
        cm.wait()
        ct.wait()

    def chunks(ids_hbm, body):
        for h in range(NCH):
            ids_v, sem_i = idbufs[h % 2]
            if h + 1 < NCH:
                nxt_v, nxt_s = idbufs[(h + 1) % 2]
                pltpu.async_copy(ids_hbm.at[pl.ds((h + 1) * CH, CH)],
                                 nxt_v, nxt_s)
            pltpu.make_async_copy(ids_hbm.at[pl.ds(0, CH)], ids_v,
                                  sem_i).wait()
            body(h, ids_v)

    def factor(f):
        load_row(f, uft_hbm, utail_hbm, uid_hbm)
        wspl = plsc.load_gather(w_v, [jnp.broadcast_to(f, (L,))])

        def ubody(h, ids_v):
            @plsc.parallel_loop(0, CH // L, unroll=4)
            def _g(v):
                idxv = ids_v[pl.ds(v * L, L)]
                r = h * CROWS + lax.shift_right_logical(v, 3)
                c = (v & 7) * L
                prod_v[r, pl.ds(c, L)] = (
                    plsc.load_gather(rowbuf, [idxv]) * wspl)
        chunks(uid_hbm, ubody)

        load_row(f, ift_hbm, itail_hbm, iid_hbm)

        def ibody(h, ids_v):
            @plsc.parallel_loop(0, CH // L, unroll=4)
            def _g(v):
                idxv = ids_v[pl.ds(v * L, L)]
                r = h * CROWS + lax.shift_right_logical(v, 3)
                c = (v & 7) * L
                prod_v[r, pl.ds(c, L)] = (
                    prod_v[r, pl.ds(c, L)]
                    * plsc.load_gather(rowbuf, [idxv]))
            pltpu.sync_copy(prod_v.at[pl.ds(h * CROWS, CROWS), :],
                            acc_sh.at[idxrows_v.at[h]], add=True)
        chunks(iid_hbm, ibody)

    factor(wid)
    factor(wid + 32)

    plsc.subcore_barrier()

    @pl.when(sid == 0)
    def _writeout():
        pltpu.sync_copy(acc_sh, accs_hbm.at[cid])


@jax.jit
def _run(user_ids, item_ids, user_factors, item_factors, w_vec, b):
    mesh = plsc.VectorSubcoreMesh(core_axis_name="c", subcore_axis_name="s")
    uft = user_factors.T
    ift = item_factors.T
    utail = jnp.pad(lax.slice(uft, (0, MAIN), (NUM_FACTORS, NUM_ROWS)),
                    ((0, 0), (0, TAILPAD - (NUM_ROWS - MAIN))))
    itail = jnp.pad(lax.slice(ift, (0, MAIN), (NUM_FACTORS, NUM_ROWS)),
                    ((0, 0), (0, TAILPAD - (NUM_ROWS - MAIN))))

    p1 = pl.kernel(
        _pass1_kernel,
        mesh=mesh,
        compiler_params=_COMPILER_PARAMS,
        out_type=jax.ShapeDtypeStruct((NC, PR, 128), jnp.float32),
        scratch_types=[
            pltpu.VMEM((ROWBUF,), jnp.float32),
            pltpu.VMEM((PR, 128), jnp.float32),
            pltpu.VMEM((CH,), jnp.int32),
            pltpu.VMEM((CH,), jnp.int32),
            pltpu.VMEM((NUM_FACTORS,), jnp.float32),
            pltpu.VMEM((8, 128), jnp.float32),
            pltpu.VMEM((NCH, CROWS), jnp.int32),
            pltpu.VMEM_SHARED((PR, 128), jnp.float32),
            pltpu.SemaphoreType.DMA,
            pltpu.SemaphoreType.DMA,
            pltpu.SemaphoreType.DMA,
            pltpu.SemaphoreType.DMA,
        ],
    )
    accs = p1(uft, ift, utail, itail, user_ids, item_ids, w_vec)
    return (accs[0] + accs[1]).reshape(BATCH) + b[0]


def kernel(user_ids, item_ids, user_factors, item_factors, W, b):
    uid = user_ids.astype(jnp.int32)
    iid = item_ids.astype(jnp.int32)
    w_vec = W.reshape(NUM_FACTORS).astype(jnp.float32)
    out = _run(uid, iid, user_factors, item_factors, w_vec,
               b.astype(jnp.float32))
    return out.reshape(BATCH, 1)
